# baseline (speedup 1.0000x reference)
import jax
import jax.numpy as jnp
from jax import lax
from jax.experimental import pallas as pl
from jax.experimental.pallas import tpu as pltpu
from jax.experimental.pallas import tpu_sc as plsc

DIM = 64
BATCH = 16384
HIST = 50

NC, NS, L = 2, 16, 16
NW = NC * NS
N = BATCH * HIST
N_PER_W = N // NW
CHUNK = 512
SUBLEN = 128
SUB = CHUNK // SUBLEN
GROUPS = CHUNK // L
N_CHUNKS = N_PER_W // CHUNK


def _maxnorm_scale(ss):
    ssc = jnp.maximum(ss, 1.0)
    i = plsc.bitcast(ssc, jnp.int32)
    y = plsc.bitcast(jnp.int32(0x5F3759DF) - (i >> 1), jnp.float32)
    h = 0.5 * ssc
    y = y * (1.5 - h * y * y)
    y = y * (1.5 - h * y * y)
    y = y * (1.5 - h * y * y)
    return jnp.where(ss > 1.0, y, 1.0)


def _body(x_hbm, table_hbm, out_hbm, idx_v, rows_v, gsem):
    wid = lax.axis_index("s") * NC + lax.axis_index("c")
    iota = lax.iota(jnp.int32, L)
    xrow_w = wid * (N_PER_W // SUBLEN)
    base_w = wid * N_PER_W

    def chunk_body(c, carry):
        r0 = base_w + c * CHUNK
        pltpu.sync_copy(x_hbm.at[pl.ds(xrow_w + c * SUB, SUB)], idx_v)
        cps = [
            pltpu.async_copy(
                table_hbm.at[idx_v.at[j]],
                rows_v.at[pl.ds(j * SUBLEN, SUBLEN)],
                gsem,
            )
            for j in range(SUB)
        ]
        for cp in cps:
            cp.wait()

        def group_body(g, gcarry):
            rows16 = g * L + iota
            acc = jnp.zeros((L,), jnp.float32)
            for d in range(DIM):
                col = jnp.full((L,), d, jnp.int32)
                v = plsc.load_gather(rows_v, [rows16, col])
                acc = acc + v * v
            scale = _maxnorm_scale(acc)
            for d in range(DIM):
                col = jnp.full((L,), d, jnp.int32)
                v = plsc.load_gather(rows_v, [rows16, col])
                plsc.store_scatter(rows_v, [rows16, col], v * scale)
            return gcarry

        lax.fori_loop(0, GROUPS, group_body, 0)
        pltpu.sync_copy(rows_v, out_hbm.at[pl.ds(r0, CHUNK)])
        return carry

    lax.fori_loop(0, N_CHUNKS, chunk_body, 0)


def kernel(x, table):
    xf = x.reshape(N // SUBLEN, SUBLEN)
    mesh = plsc.VectorSubcoreMesh(core_axis_name="c", subcore_axis_name="s")
    out = pl.kernel(
        _body,
        out_type=jax.ShapeDtypeStruct((N, DIM), jnp.float32),
        mesh=mesh,
        compiler_params=pltpu.CompilerParams(
            use_tc_tiling_on_sc=False, needs_layout_passes=False
        ),
        scratch_types=[
            pltpu.VMEM((SUB, SUBLEN), jnp.int32),
            pltpu.VMEM((CHUNK, DIM), jnp.float32),
            pltpu.SemaphoreType.DMA,
        ],
    )(xf, table)
    return out.reshape(BATCH, HIST, DIM)


# speedup vs baseline: 1.9731x; 1.9731x over previous
import jax
import jax.numpy as jnp
from jax import lax
from jax.experimental import pallas as pl
from jax.experimental.pallas import tpu as pltpu
from jax.experimental.pallas import tpu_sc as plsc

DIM = 64
BATCH = 16384
HIST = 50

NC, NS, L = 2, 16, 16
NW = NC * NS
N = BATCH * HIST
N_PER_W = N // NW
CHUNK = 512
SUBLEN = 128
SUB = CHUNK // SUBLEN
GROUPS = CHUNK // L
N_CHUNKS = N_PER_W // CHUNK


def _maxnorm_scale(ss):
    ssc = jnp.maximum(ss, 1.0)
    i = plsc.bitcast(ssc, jnp.int32)
    y = plsc.bitcast(jnp.int32(0x5F3759DF) - (i >> 1), jnp.float32)
    h = 0.5 * ssc
    y = y * (1.5 - h * y * y)
    y = y * (1.5 - h * y * y)
    y = y * (1.5 - h * y * y)
    return jnp.where(ss > 1.0, y, 1.0)


def _body(x_hbm, table_hbm, out_hbm, idx_v, rows_v, gsem):
    wid = lax.axis_index("s") * NC + lax.axis_index("c")
    iota = lax.iota(jnp.int32, L)
    xrow_w = wid * (N_PER_W // SUBLEN)
    base_w = wid * N_PER_W

    def chunk_body(c, carry):
        r0 = base_w + c * CHUNK
        pltpu.sync_copy(x_hbm.at[pl.ds(xrow_w + c * SUB, SUB)], idx_v)
        cps = [
            pltpu.async_copy(
                table_hbm.at[idx_v.at[j]],
                rows_v.at[pl.ds(j * SUBLEN, SUBLEN)],
                gsem,
            )
            for j in range(SUB)
        ]
        for cp in cps:
            cp.wait()

        def group_body(g, gcarry):
            rows16 = g * L + iota
            acc = jnp.zeros((L,), jnp.float32)
            col = iota
            for d in range(DIM):
                v = plsc.load_gather(rows_v, [rows16, col])
                acc = acc + v * v
                col = (col + 1) & (DIM - 1)
            scale = _maxnorm_scale(acc)
            col = iota
            for d in range(DIM):
                v = plsc.load_gather(rows_v, [rows16, col])
                plsc.store_scatter(rows_v, [rows16, col], v * scale)
                col = (col + 1) & (DIM - 1)
            return gcarry

        lax.fori_loop(0, GROUPS, group_body, 0)
        pltpu.sync_copy(rows_v, out_hbm.at[pl.ds(r0, CHUNK)])
        return carry

    lax.fori_loop(0, N_CHUNKS, chunk_body, 0)


def kernel(x, table):
    xf = x.reshape(N // SUBLEN, SUBLEN)
    mesh = plsc.VectorSubcoreMesh(core_axis_name="c", subcore_axis_name="s")
    out = pl.kernel(
        _body,
        out_type=jax.ShapeDtypeStruct((N, DIM), jnp.float32),
        mesh=mesh,
        compiler_params=pltpu.CompilerParams(
            use_tc_tiling_on_sc=False, needs_layout_passes=False
        ),
        scratch_types=[
            pltpu.VMEM((SUB, SUBLEN), jnp.int32),
            pltpu.VMEM((CHUNK, DIM), jnp.float32),
            pltpu.SemaphoreType.DMA,
        ],
    )(xf, table)
    return out.reshape(BATCH, HIST, DIM)
